# baseline (speedup 1.0000x reference)
import jax
import jax.numpy as jnp
from jax import lax
from jax.experimental import pallas as pl
from jax.experimental.pallas import tpu as pltpu
from jax.experimental.pallas import tpu_sc as plsc

def _take16(a, idx):
    return lax.gather(
        a, idx[:, None],
        dimension_numbers=lax.GatherDimensionNumbers(
            offset_dims=(), collapsed_slice_dims=(0,), start_index_map=(0,)),
        slice_sizes=(1,),
        mode=lax.GatherScatterMode.PROMISE_IN_BOUNDS)


M = 100000
D = 128
B = 16384
CAP = 4000
NWORK = M // CAP
LISTPAD = 160
KSHIFT = 14


def _body(bimg, blab, brep, blast, keys, vals, xs, ys,
          oimg, olab, orep, olast,
          keys_v, vals_v, y_v, win_v, lab_v, rep_v, last_v, dl_v, vl_v,
          row_v, sem_img, sem_g, sem_s):
    cid = lax.axis_index("c")
    sid = lax.axis_index("s")
    wid = sid * 2 + cid

    @pl.when(wid < NWORK)
    def _work():
        base = wid * CAP

        cp_img = pltpu.async_copy(bimg.at[pl.ds(base, CAP)],
                                  oimg.at[pl.ds(base, CAP)], sem_img)

        pltpu.sync_copy(keys, keys_v)
        pltpu.sync_copy(vals, vals_v)
        pltpu.sync_copy(ys, y_v)
        pltpu.sync_copy(blab.at[pl.ds(base, CAP)], lab_v)
        pltpu.sync_copy(brep.at[pl.ds(base, CAP)], rep_v)
        pltpu.sync_copy(blast.at[pl.ds(base, CAP)], last_v)

        lanes = lax.iota(jnp.int32, 16)
        neg1 = jnp.full((16,), -1, jnp.int32)
        zero16 = jnp.zeros((16,), jnp.int32)

        def init_body(j, c):
            win_v[pl.ds(j * 16, 16)] = neg1
            return c
        lax.fori_loop(0, CAP // 16, init_body, 0)

        def scan_body(j, c):
            k = keys_v[pl.ds(j * 16, 16)]
            comb = (k << KSHIFT) | (lanes + j * 16)
            sc = lax.sort(comb, dimension=0)
            sk = sc >> KSHIFT
            sp = sc & ((1 << KSHIFT) - 1)
            nxt = _take16(sk, jnp.minimum(lanes + 1, 15))
            keep = (nxt != sk) | (lanes == 15)
            inr = (sk >= base) & (sk < base + CAP)
            valid = keep & inr
            loc = jnp.where(valid, sk - base, 0)
            cur = plsc.load_gather(win_v, [loc])
            plsc.store_scatter(win_v, [loc], jnp.maximum(cur, sp), mask=valid)
            return c
        lax.fori_loop(0, B // 16, scan_body, 0)

        def comp_body(j, cnt):
            w = win_v[pl.ds(j * 16, 16)]
            m = w >= 0
            r = lanes + j * 16
            wsafe = jnp.where(m, w, 0)
            val = plsc.load_gather(vals_v, [wsafe])
            lab = plsc.load_gather(y_v, [jnp.where(m, val, 0)])
            plsc.store_scatter(lab_v, [r], lab, mask=m)
            plsc.store_scatter(rep_v, [r], zero16, mask=m)
            plsc.store_scatter(last_v, [r], zero16, mask=m)
            plsc.store_compressed(dl_v.at[pl.ds(cnt, 16)], r + base, mask=m)
            plsc.store_compressed(vl_v.at[pl.ds(cnt, 16)], val, mask=m)
            return cnt + jnp.sum(jnp.where(m, 1, 0))
        cnt = lax.fori_loop(0, CAP // 16, comp_body, jnp.int32(0))

        @pl.when(cnt > 0)
        def _pad():
            zi = jnp.zeros((16,), jnp.int32)
            d0 = _take16(dl_v[pl.ds(0, 16)], zi)
            v0 = _take16(vl_v[pl.ds(0, 16)], zi)

            def pad_body(p, c):
                idxs = cnt + p * 16 + lanes
                plsc.store_scatter(dl_v, [idxs], d0)
                plsc.store_scatter(vl_v, [idxs], v0)
                return c
            lax.fori_loop(0, 8, pad_body, 0)

        cp_img.wait()

        nsup = (cnt + 127) // 128

        def sup_body(s, c):
            gws = []
            for b in range(8):
                kk = s * 8 + b
                vvec = vl_v[pl.ds(kk * 16, 16)]
                gws.append(pltpu.async_copy(xs.at[vvec], row_v.at[b], sem_g))
            for g in gws:
                g.wait()
            sws = []
            for b in range(8):
                kk = s * 8 + b
                dvec = dl_v[pl.ds(kk * 16, 16)]
                sws.append(pltpu.async_copy(row_v.at[b], oimg.at[dvec], sem_s))
            for sw in sws:
                sw.wait()
            return c
        lax.fori_loop(0, nsup, sup_body, 0)

        pltpu.sync_copy(lab_v, olab.at[pl.ds(base, CAP)])
        pltpu.sync_copy(rep_v, orep.at[pl.ds(base, CAP)])
        pltpu.sync_copy(last_v, olast.at[pl.ds(base, CAP)])


_mesh = plsc.VectorSubcoreMesh(core_axis_name="c", subcore_axis_name="s")

_sc_overwrite = pl.kernel(
    _body,
    out_type=(
        jax.ShapeDtypeStruct((M, D), jnp.float32),
        jax.ShapeDtypeStruct((M,), jnp.int32),
        jax.ShapeDtypeStruct((M,), jnp.int32),
        jax.ShapeDtypeStruct((M,), jnp.int32),
    ),
    mesh=_mesh,
    compiler_params=pltpu.CompilerParams(needs_layout_passes=False),
    scratch_types=(
        pltpu.VMEM((B,), jnp.int32),
        pltpu.VMEM((B,), jnp.int32),
        pltpu.VMEM((B,), jnp.int32),
        pltpu.VMEM((CAP,), jnp.int32),
        pltpu.VMEM((CAP,), jnp.int32),
        pltpu.VMEM((CAP,), jnp.int32),
        pltpu.VMEM((CAP,), jnp.int32),
        pltpu.VMEM((CAP + LISTPAD,), jnp.int32),
        pltpu.VMEM((CAP + LISTPAD,), jnp.int32),
        pltpu.VMEM((8, 16, D), jnp.float32),
        pltpu.SemaphoreType.DMA,
        pltpu.SemaphoreType.DMA,
        pltpu.SemaphoreType.DMA,
    ),
)


def kernel(buffer_img, buffer_label, buffer_replay_times, buffer_last_replay,
           idx_keys, idx_vals, x, y):
    out_img, out_label, out_rep, out_last = _sc_overwrite(
        buffer_img,
        buffer_label.astype(jnp.int32),
        buffer_replay_times.astype(jnp.int32),
        buffer_last_replay.astype(jnp.int32),
        idx_keys.astype(jnp.int32),
        idx_vals.astype(jnp.int32),
        x,
        y.astype(jnp.int32))
    return (out_img,
            out_label.astype(buffer_label.dtype),
            out_rep.astype(buffer_replay_times.dtype),
            out_last.astype(buffer_last_replay.dtype))


# speedup vs baseline: 1.0003x; 1.0003x over previous
import jax
import jax.numpy as jnp
from jax import lax
from jax.experimental import pallas as pl
from jax.experimental.pallas import tpu as pltpu
from jax.experimental.pallas import tpu_sc as plsc

def _take16(a, idx):
    return lax.gather(
        a, idx[:, None],
        dimension_numbers=lax.GatherDimensionNumbers(
            offset_dims=(), collapsed_slice_dims=(0,), start_index_map=(0,)),
        slice_sizes=(1,),
        mode=lax.GatherScatterMode.PROMISE_IN_BOUNDS)


M = 100000
D = 128
B = 16384
CAP = 4000
NWORK = M // CAP
LISTPAD = 160
KSHIFT = 14


def _body(bimg, blab, brep, blast, keys, vals, xs, ys,
          oimg, olab, orep, olast,
          keys_v, vals_v, y_v, win_v, lab_v, rep_v, last_v, dl_v, vl_v,
          row_v, sem_img, sem_g, sem_s):
    cid = lax.axis_index("c")
    sid = lax.axis_index("s")
    wid = sid * 2 + cid

    @pl.when(wid < NWORK)
    def _work():
        base = wid * CAP

        cp_img = pltpu.async_copy(bimg.at[pl.ds(base, CAP)],
                                  oimg.at[pl.ds(base, CAP)], sem_img)

        with jax.named_scope("ph_stage_in"):
            pltpu.sync_copy(keys, keys_v)
            pltpu.sync_copy(vals, vals_v)
            pltpu.sync_copy(ys, y_v)
            pltpu.sync_copy(blab.at[pl.ds(base, CAP)], lab_v)
            pltpu.sync_copy(brep.at[pl.ds(base, CAP)], rep_v)
            pltpu.sync_copy(blast.at[pl.ds(base, CAP)], last_v)

        lanes = lax.iota(jnp.int32, 16)
        neg1 = jnp.full((16,), -1, jnp.int32)
        zero16 = jnp.zeros((16,), jnp.int32)

        def init_body(j, c):
            win_v[pl.ds(j * 16, 16)] = neg1
            return c
        lax.fori_loop(0, CAP // 16, init_body, 0)

        def scan_body(j, c):
            k = keys_v[pl.ds(j * 16, 16)]
            comb = (k << KSHIFT) | (lanes + j * 16)
            sc = lax.sort(comb, dimension=0)
            sk = sc >> KSHIFT
            sp = sc & ((1 << KSHIFT) - 1)
            nxt = _take16(sk, jnp.minimum(lanes + 1, 15))
            keep = (nxt != sk) | (lanes == 15)
            inr = (sk >= base) & (sk < base + CAP)
            valid = keep & inr
            loc = jnp.where(valid, sk - base, 0)
            cur = plsc.load_gather(win_v, [loc])
            plsc.store_scatter(win_v, [loc], jnp.maximum(cur, sp), mask=valid)
            return c
        with jax.named_scope("ph_scan"):
            lax.fori_loop(0, B // 16, scan_body, 0)

        def comp_body(j, cnt):
            w = win_v[pl.ds(j * 16, 16)]
            m = w >= 0
            r = lanes + j * 16
            wsafe = jnp.where(m, w, 0)
            val = plsc.load_gather(vals_v, [wsafe])
            lab = plsc.load_gather(y_v, [jnp.where(m, val, 0)])
            plsc.store_scatter(lab_v, [r], lab, mask=m)
            plsc.store_scatter(rep_v, [r], zero16, mask=m)
            plsc.store_scatter(last_v, [r], zero16, mask=m)
            plsc.store_compressed(dl_v.at[pl.ds(cnt, 16)], r + base, mask=m)
            plsc.store_compressed(vl_v.at[pl.ds(cnt, 16)], val, mask=m)
            return cnt + jnp.sum(jnp.where(m, 1, 0))
        with jax.named_scope("ph_compact"):
            cnt = lax.fori_loop(0, CAP // 16, comp_body, jnp.int32(0))

        @pl.when(cnt > 0)
        def _pad():
            zi = jnp.zeros((16,), jnp.int32)
            d0 = _take16(dl_v[pl.ds(0, 16)], zi)
            v0 = _take16(vl_v[pl.ds(0, 16)], zi)

            def pad_body(p, c):
                idxs = cnt + p * 16 + lanes
                plsc.store_scatter(dl_v, [idxs], d0)
                plsc.store_scatter(vl_v, [idxs], v0)
                return c
            lax.fori_loop(0, 8, pad_body, 0)

        with jax.named_scope("ph_imgcopy_wait"):
            cp_img.wait()

        nsup = (cnt + 127) // 128

        def sup_body(s, c):
            gws = []
            for b in range(8):
                kk = s * 8 + b
                vvec = vl_v[pl.ds(kk * 16, 16)]
                gws.append(pltpu.async_copy(xs.at[vvec], row_v.at[b], sem_g))
            for g in gws:
                g.wait()
            sws = []
            for b in range(8):
                kk = s * 8 + b
                dvec = dl_v[pl.ds(kk * 16, 16)]
                sws.append(pltpu.async_copy(row_v.at[b], oimg.at[dvec], sem_s))
            for sw in sws:
                sw.wait()
            return c
        with jax.named_scope("ph_rowdma"):
            lax.fori_loop(0, nsup, sup_body, 0)

        with jax.named_scope("ph_stage_out"):
            pltpu.sync_copy(lab_v, olab.at[pl.ds(base, CAP)])
            pltpu.sync_copy(rep_v, orep.at[pl.ds(base, CAP)])
            pltpu.sync_copy(last_v, olast.at[pl.ds(base, CAP)])


_mesh = plsc.VectorSubcoreMesh(core_axis_name="c", subcore_axis_name="s")

_sc_overwrite = pl.kernel(
    _body,
    out_type=(
        jax.ShapeDtypeStruct((M, D), jnp.float32),
        jax.ShapeDtypeStruct((M,), jnp.int32),
        jax.ShapeDtypeStruct((M,), jnp.int32),
        jax.ShapeDtypeStruct((M,), jnp.int32),
    ),
    mesh=_mesh,
    compiler_params=pltpu.CompilerParams(needs_layout_passes=False),
    scratch_types=(
        pltpu.VMEM((B,), jnp.int32),
        pltpu.VMEM((B,), jnp.int32),
        pltpu.VMEM((B,), jnp.int32),
        pltpu.VMEM((CAP,), jnp.int32),
        pltpu.VMEM((CAP,), jnp.int32),
        pltpu.VMEM((CAP,), jnp.int32),
        pltpu.VMEM((CAP,), jnp.int32),
        pltpu.VMEM((CAP + LISTPAD,), jnp.int32),
        pltpu.VMEM((CAP + LISTPAD,), jnp.int32),
        pltpu.VMEM((8, 16, D), jnp.float32),
        pltpu.SemaphoreType.DMA,
        pltpu.SemaphoreType.DMA,
        pltpu.SemaphoreType.DMA,
    ),
)


def kernel(buffer_img, buffer_label, buffer_replay_times, buffer_last_replay,
           idx_keys, idx_vals, x, y):
    out_img, out_label, out_rep, out_last = _sc_overwrite(
        buffer_img,
        buffer_label.astype(jnp.int32),
        buffer_replay_times.astype(jnp.int32),
        buffer_last_replay.astype(jnp.int32),
        idx_keys.astype(jnp.int32),
        idx_vals.astype(jnp.int32),
        x,
        y.astype(jnp.int32))
    return (out_img,
            out_label.astype(buffer_label.dtype),
            out_rep.astype(buffer_replay_times.dtype),
            out_last.astype(buffer_last_replay.dtype))


# speedup vs baseline: 13.2857x; 13.2815x over previous
import jax
import jax.numpy as jnp
from jax import lax
from jax.experimental import pallas as pl
from jax.experimental.pallas import tpu as pltpu
from jax.experimental.pallas import tpu_sc as plsc


def _take16(a, idx):
    return lax.gather(
        a, idx[:, None],
        dimension_numbers=lax.GatherDimensionNumbers(
            offset_dims=(), collapsed_slice_dims=(0,), start_index_map=(0,)),
        slice_sizes=(1,),
        mode=lax.GatherScatterMode.PROMISE_IN_BOUNDS)


M = 100000
D = 128
B = 16384
CAP = 4000
NWORK = M // CAP
LISTPAD = 160
KSHIFT = 14


def _body(img_r, lab_r, rep_r, last_r, keys, vals, xs, ys,
          keys_v, vals_v, y_v, win_v, lab_v, rep_v, last_v, dl_v, vl_v,
          row_v, sem_g, sem_s):
    cid = lax.axis_index("c")
    sid = lax.axis_index("s")
    wid = sid * 2 + cid

    @pl.when(wid < NWORK)
    def _work():
        base = wid * CAP

        with jax.named_scope("ph_stage_in"):
            pltpu.sync_copy(keys, keys_v)
            pltpu.sync_copy(vals, vals_v)
            pltpu.sync_copy(ys, y_v)
            pltpu.sync_copy(lab_r.at[pl.ds(base, CAP)], lab_v)
            pltpu.sync_copy(rep_r.at[pl.ds(base, CAP)], rep_v)
            pltpu.sync_copy(last_r.at[pl.ds(base, CAP)], last_v)

        lanes = lax.iota(jnp.int32, 16)
        neg1 = jnp.full((16,), -1, jnp.int32)
        zero16 = jnp.zeros((16,), jnp.int32)

        def init_body(j, c):
            win_v[pl.ds(j * 16, 16)] = neg1
            return c
        lax.fori_loop(0, CAP // 16, init_body, 0)

        def scan_body(j, c):
            k = keys_v[pl.ds(j * 16, 16)]
            comb = (k << KSHIFT) | (lanes + j * 16)
            sc = lax.sort(comb, dimension=0)
            sk = sc >> KSHIFT
            sp = sc & ((1 << KSHIFT) - 1)
            nxt = _take16(sk, jnp.minimum(lanes + 1, 15))
            keep = (nxt != sk) | (lanes == 15)
            inr = (sk >= base) & (sk < base + CAP)
            valid = keep & inr
            loc = jnp.where(valid, sk - base, 0)
            cur = plsc.load_gather(win_v, [loc])
            plsc.store_scatter(win_v, [loc], jnp.maximum(cur, sp), mask=valid)
            return c
        with jax.named_scope("ph_scan"):
            lax.fori_loop(0, B // 16, scan_body, 0)

        def comp_body(j, cnt):
            w = win_v[pl.ds(j * 16, 16)]
            m = w >= 0
            r = lanes + j * 16
            wsafe = jnp.where(m, w, 0)
            val = plsc.load_gather(vals_v, [wsafe])
            lab = plsc.load_gather(y_v, [jnp.where(m, val, 0)])
            plsc.store_scatter(lab_v, [r], lab, mask=m)
            plsc.store_scatter(rep_v, [r], zero16, mask=m)
            plsc.store_scatter(last_v, [r], zero16, mask=m)
            plsc.store_compressed(dl_v.at[pl.ds(cnt, 16)], r + base, mask=m)
            plsc.store_compressed(vl_v.at[pl.ds(cnt, 16)], val, mask=m)
            return cnt + jnp.sum(jnp.where(m, 1, 0))
        with jax.named_scope("ph_compact"):
            cnt = lax.fori_loop(0, CAP // 16, comp_body, jnp.int32(0))

        @pl.when(cnt > 0)
        def _pad():
            zi = jnp.zeros((16,), jnp.int32)
            d0 = _take16(dl_v[pl.ds(0, 16)], zi)
            v0 = _take16(vl_v[pl.ds(0, 16)], zi)

            def pad_body(p, c):
                idxs = cnt + p * 16 + lanes
                plsc.store_scatter(dl_v, [idxs], d0)
                plsc.store_scatter(vl_v, [idxs], v0)
                return c
            lax.fori_loop(0, 8, pad_body, 0)

        nsup = (cnt + 127) // 128

        def sup_body(s, c):
            gws = []
            for b in range(8):
                kk = s * 8 + b
                vvec = vl_v[pl.ds(kk * 16, 16)]
                gws.append(pltpu.async_copy(xs.at[vvec], row_v.at[b], sem_g))
            for g in gws:
                g.wait()
            sws = []
            for b in range(8):
                kk = s * 8 + b
                dvec = dl_v[pl.ds(kk * 16, 16)]
                sws.append(pltpu.async_copy(row_v.at[b], img_r.at[dvec], sem_s))
            for sw in sws:
                sw.wait()
            return c
        with jax.named_scope("ph_rowdma"):
            lax.fori_loop(0, nsup, sup_body, 0)

        with jax.named_scope("ph_stage_out"):
            pltpu.sync_copy(lab_v, lab_r.at[pl.ds(base, CAP)])
            pltpu.sync_copy(rep_v, rep_r.at[pl.ds(base, CAP)])
            pltpu.sync_copy(last_v, last_r.at[pl.ds(base, CAP)])


_mesh = plsc.VectorSubcoreMesh(core_axis_name="c", subcore_axis_name="s")

_sc_overwrite = pl.kernel(
    _body,
    out_type=(),
    mesh=_mesh,
    compiler_params=pltpu.CompilerParams(needs_layout_passes=False),
    scratch_types=(
        pltpu.VMEM((B,), jnp.int32),
        pltpu.VMEM((B,), jnp.int32),
        pltpu.VMEM((B,), jnp.int32),
        pltpu.VMEM((CAP,), jnp.int32),
        pltpu.VMEM((CAP,), jnp.int32),
        pltpu.VMEM((CAP,), jnp.int32),
        pltpu.VMEM((CAP,), jnp.int32),
        pltpu.VMEM((CAP + LISTPAD,), jnp.int32),
        pltpu.VMEM((CAP + LISTPAD,), jnp.int32),
        pltpu.VMEM((8, 16, D), jnp.float32),
        pltpu.SemaphoreType.DMA,
        pltpu.SemaphoreType.DMA,
    ),
)


def kernel(buffer_img, buffer_label, buffer_replay_times, buffer_last_replay,
           idx_keys, idx_vals, x, y):
    img_ref = jax.new_ref(buffer_img)
    lab_ref = jax.new_ref(buffer_label.astype(jnp.int32))
    rep_ref = jax.new_ref(buffer_replay_times.astype(jnp.int32))
    last_ref = jax.new_ref(buffer_last_replay.astype(jnp.int32))
    _sc_overwrite(img_ref, lab_ref, rep_ref, last_ref,
                  idx_keys.astype(jnp.int32),
                  idx_vals.astype(jnp.int32),
                  x,
                  y.astype(jnp.int32))
    return (jax.freeze(img_ref),
            jax.freeze(lab_ref).astype(buffer_label.dtype),
            jax.freeze(rep_ref).astype(buffer_replay_times.dtype),
            jax.freeze(last_ref).astype(buffer_last_replay.dtype))
